# baseline (speedup 1.0000x reference)
import functools

import jax
import jax.numpy as jnp
from jax import lax
from jax.experimental import pallas as pl
from jax.experimental.pallas import tpu as pltpu
from jax.experimental.pallas import tpu_sc as plsc

_NC = 2
_NS = 16
_NW = _NC * _NS
_L = 16
_B = 16384
_F = 64
_BPW = _B // _NW
_ROWS = _B + _L
_MCAP = 2048
_CB = 4
_CW = _CB * 128
_SD = 4

_mesh = plsc.VectorSubcoreMesh(core_axis_name="c", subcore_axis_name="s")

_GD = lax.GatherDimensionNumbers(
    offset_dims=(), collapsed_slice_dims=(0,), start_index_map=(0,))


def _splat(vec, j):
    idx = jnp.full((_L, 1), j, jnp.int32)
    return lax.gather(vec, idx, _GD, (1,),
                      mode=lax.GatherScatterMode.PROMISE_IN_BOUNDS)


def _make_k1(n_ids, bias_lane):
    cols_tot = -(-n_ids // 128)
    per = cols_tot // _NW
    extra = cols_tot - per * _NW

    @functools.partial(
        pl.kernel,
        out_type=jax.ShapeDtypeStruct((_ROWS, 128), jnp.float32),
        mesh=_mesh,
        compiler_params=pltpu.CompilerParams(
            needs_layout_passes=False, use_tc_tiling_on_sc=True),
        scratch_types=[
            pltpu.VMEM((_B,), jnp.int32),
            pltpu.VMEM((_MCAP + _L,), jnp.int32),
            pltpu.VMEM((_MCAP + _L,), jnp.int32),
            pltpu.VMEM((2, _F, _CW), jnp.float32),
            pltpu.VMEM((2, 1, _CW), jnp.float32),
            pltpu.VMEM((_SD, _L, 128), jnp.float32),
            pltpu.VMEM((128, _L), jnp.float32),
            pltpu.VMEM((_SD, _L), jnp.int32),
            pltpu.SMEM((1,), jnp.int32),
            pltpu.SemaphoreType.DMA,
            pltpu.SemaphoreType.DMA,
            pltpu.SemaphoreType.DMA,
        ],
    )
    def k1(ids_hbm, tab_hbm, bias_hbm, rows_hbm,
           ids_v, muid_v, mexi_v, chunk_v, bch_v, stg_v, stgt_v, idx2_v,
           fired_s, semo, semt, semb):
        wid = lax.axis_index("s") * _NC + lax.axis_index("c")
        cbase = wid * per + jnp.minimum(wid, extra)
        ccnt = jnp.where(wid < extra, per + 1, per)
        lo = cbase * 128
        hi = (cbase + ccnt) * 128

        lanes = lax.iota(jnp.int32, _L)
        zeros16 = jnp.zeros((_L,), jnp.float32)
        ones16 = jnp.ones((_L,), jnp.float32)

        for s in range(_SD):
            for j in range(_L):
                for q in range(4, 8):
                    stg_v[s, j, pl.ds(q * _L, _L)] = zeros16
        for f in range(66, 128):
            stgt_v[f, pl.ds(0, _L)] = zeros16
        stgt_v[65, pl.ds(0, _L)] = ones16
        fired_s[0] = 0

        pltpu.sync_copy(ids_hbm, ids_v)

        def scan(g, off):
            idv = ids_v[pl.ds(g * _L, _L)]
            m = (idv >= lo) & (idv < hi)
            plsc.store_compressed(muid_v.at[pl.ds(off, _L)], idv, mask=m)
            exv = g * _L + lanes
            plsc.store_compressed(mexi_v.at[pl.ds(off, _L)], exv, mask=m)
            cnt = plsc.all_reduce_population_count(m)[0]
            return jnp.minimum(off + cnt, _MCAP)

        mtot = lax.fori_loop(0, _B // _L, scan, 0)
        ngrp = (mtot + _L - 1) // _L

        nchunk = (ccnt + _CB - 1) // _CB

        def col_off(t):
            return pl.multiple_of(
                jnp.minimum(cbase + t * _CB, cols_tot - _CB) * 128, 128)

        def fire_chunk(t):
            buf = lax.rem(t, 2)
            coff = col_off(t)
            pltpu.async_copy(
                tab_hbm.at[:, pl.ds(coff, _CW)], chunk_v.at[buf], semt)
            pltpu.async_copy(
                bias_hbm.at[pl.ds(0, 1), pl.ds(coff, _CW)], bch_v.at[buf],
                semb)

        def drain_one_scatter():
            pltpu.make_async_copy(
                stg_v.at[0], rows_hbm.at[idx2_v.at[0]], semo).wait()

        fire_chunk(0)

        def chunk(t, carry):
            buf = lax.rem(t, 2)
            coff = col_off(t)
            cb_end = coff + _CW

            @pl.when(t + 1 < nchunk)
            def _():
                fire_chunk(t + 1)

            pltpu.make_async_copy(
                tab_hbm.at[:, pl.ds(0, _CW)], chunk_v.at[0], semt).wait()
            pltpu.make_async_copy(
                bias_hbm.at[pl.ds(0, 1), pl.ds(0, _CW)], bch_v.at[0],
                semb).wait()

            def group(g, carry2):
                muidg = muid_v[pl.ds(g * _L, _L)]
                mexig = mexi_v[pl.ds(g * _L, _L)]
                valid = (g * _L + lanes) < mtot
                inch = valid & (muidg >= coff) & (muidg < cb_end)
                cnt = plsc.all_reduce_population_count(inch)[0]

                @pl.when(cnt > 0)
                def _():
                    f = fired_s[0]

                    @pl.when(f >= _SD)
                    def _():
                        drain_one_scatter()

                    s = lax.rem(f, _SD)
                    cols = jnp.where(inch, muidg - coff, 0)
                    safe_ex = jnp.where(inch, mexig, _B + lanes)
                    idx2_v[s, pl.ds(0, _L)] = safe_ex
                    for ff in range(_F):
                        stgt_v[ff, pl.ds(0, _L)] = plsc.load_gather(
                            chunk_v.at[buf],
                            [jnp.full((_L,), ff, jnp.int32), cols])
                    stgt_v[bias_lane, pl.ds(0, _L)] = plsc.load_gather(
                        bch_v.at[buf, 0], [cols])
                    stgt_v[129 - bias_lane, pl.ds(0, _L)] = ones16
                    for j in range(_L):
                        cj = jnp.full((_L,), j, jnp.int32)
                        for q in range(5):
                            stg_v[s, j, pl.ds(q * _L, _L)] = plsc.load_gather(
                                stgt_v, [q * _L + lanes, cj])
                    pltpu.async_copy(
                        stg_v.at[s], rows_hbm.at[idx2_v.at[s]], semo)
                    fired_s[0] = f + 1

                return carry2

            lax.fori_loop(0, ngrp, group, 0)
            return carry

        lax.fori_loop(0, nchunk, chunk, 0)

        def final_drain(i, carry):
            drain_one_scatter()
            return carry

        lax.fori_loop(0, jnp.minimum(fired_s[0], _SD), final_drain, 0)

    return k1


_k1_user = _make_k1(1000000, 64)
_k1_item = _make_k1(100000, 65)


@functools.partial(
    pl.kernel,
    out_type=jax.ShapeDtypeStruct((_B,), jnp.float32),
    mesh=_mesh,
    compiler_params=pltpu.CompilerParams(
        needs_layout_passes=False, use_tc_tiling_on_sc=True),
    scratch_types=[
        pltpu.VMEM((128, 128), jnp.float32),
        pltpu.VMEM((128, 128), jnp.float32),
        pltpu.VMEM((_BPW,), jnp.float32),
    ],
)
def _k2(ue_rows_hbm, ie_rows_hbm, out_hbm, ue_v, ie_v, out_v):
    wid = lax.axis_index("s") * _NC + lax.axis_index("c")
    base = wid * _BPW
    lanes = lax.iota(jnp.int32, _L)

    def block(b, carry):
        e0 = base + b * 128
        pltpu.sync_copy(ue_rows_hbm.at[pl.ds(e0, 128), :], ue_v)
        pltpu.sync_copy(ie_rows_hbm.at[pl.ds(e0, 128), :], ie_v)

        def grp(g, carry2):
            res = jnp.zeros((_L,), jnp.float32)
            for j in range(_L):
                accs = []
                for q in range(8):
                    sl = pl.ds(q * _L, _L)
                    accs.append(ue_v[g * _L + j, sl] * ie_v[g * _L + j, sl])
                p = (((accs[0] + accs[1]) + (accs[2] + accs[3]))
                     + ((accs[4] + accs[5]) + (accs[6] + accs[7])))
                res = jnp.where(lanes == j, jnp.sum(p), res)
            out_v[pl.ds(b * 128 + g * _L, _L)] = res
            return carry2

        lax.fori_loop(0, 8, grp, 0)
        return carry

    lax.fori_loop(0, _BPW // 128, block, 0)
    pltpu.sync_copy(out_v, out_hbm.at[pl.ds(base, _BPW)])


def kernel(users, items, user_embeddings, item_embeddings, user_biases,
           item_biases):
    uet = user_embeddings.T
    iet = item_embeddings.T
    ubT = user_biases.T
    ibT = item_biases.T
    ue_rows = _k1_user(users, uet, ubT)
    ie_rows = _k1_item(items, iet, ibT)
    out = _k2(ue_rows, ie_rows)
    return out.reshape(_B, 1)


# speedup vs baseline: 1.8540x; 1.8540x over previous
import functools

import jax
import jax.numpy as jnp
from jax import lax
from jax.experimental import pallas as pl
from jax.experimental.pallas import tpu as pltpu
from jax.experimental.pallas import tpu_sc as plsc

_NC = 2
_NS = 16
_NW = _NC * _NS
_L = 16
_B = 16384
_F = 64
_BPW = _B // _NW

_mesh = plsc.VectorSubcoreMesh(core_axis_name="c", subcore_axis_name="s")
_params = pltpu.CompilerParams(
    needs_layout_passes=False, use_tc_tiling_on_sc=False)


@functools.partial(
    pl.kernel,
    out_type=(jax.ShapeDtypeStruct((_B, _F), jnp.float32),
              jax.ShapeDtypeStruct((_B,), jnp.float32)),
    mesh=_mesh,
    compiler_params=_params,
    scratch_types=[
        pltpu.VMEM((_BPW,), jnp.int32),
        pltpu.VMEM((_BPW, _F), jnp.float32),
        pltpu.VMEM((_BPW,), jnp.float32),
        pltpu.SemaphoreType.DMA,
        pltpu.SemaphoreType.DMA,
    ],
)
def _gather_rows(ids_hbm, tab_hbm, bias_hbm, rows_hbm, bg_hbm,
                 idx_v, rows_v, bv_v, semr, semb):
    wid = lax.axis_index("s") * _NC + lax.axis_index("c")
    base = wid * _BPW
    pltpu.sync_copy(ids_hbm.at[pl.ds(base, _BPW)], idx_v)
    cpr = pltpu.async_copy(tab_hbm.at[idx_v], rows_v, semr)
    cpb = pltpu.async_copy(bias_hbm.at[idx_v], bv_v, semb)
    cpr.wait()
    cpb.wait()
    pltpu.sync_copy(rows_v, rows_hbm.at[pl.ds(base, _BPW), :])
    pltpu.sync_copy(bv_v, bg_hbm.at[pl.ds(base, _BPW)])


@functools.partial(
    pl.kernel,
    out_type=jax.ShapeDtypeStruct((_B,), jnp.float32),
    mesh=_mesh,
    compiler_params=_params,
    scratch_types=[
        pltpu.VMEM((_BPW, _F), jnp.float32),
        pltpu.VMEM((_BPW, _F), jnp.float32),
        pltpu.VMEM((_BPW,), jnp.float32),
        pltpu.VMEM((_BPW,), jnp.float32),
        pltpu.VMEM((_BPW,), jnp.float32),
        pltpu.SemaphoreType.DMA,
        pltpu.SemaphoreType.DMA,
    ],
)
def _combine(ue_hbm, ub_hbm, ie_hbm, ib_hbm, out_hbm,
             ue_v, ie_v, ub_v, ib_v, out_v, sem0, sem1):
    wid = lax.axis_index("s") * _NC + lax.axis_index("c")
    base = wid * _BPW
    cp0 = pltpu.async_copy(ue_hbm.at[pl.ds(base, _BPW), :], ue_v, sem0)
    cp1 = pltpu.async_copy(ie_hbm.at[pl.ds(base, _BPW), :], ie_v, sem1)
    pltpu.sync_copy(ub_hbm.at[pl.ds(base, _BPW)], ub_v)
    pltpu.sync_copy(ib_hbm.at[pl.ds(base, _BPW)], ib_v)
    cp0.wait()
    cp1.wait()

    lanes = lax.iota(jnp.int32, _L)

    def group(g, carry):
        res = jnp.zeros((_L,), jnp.float32)
        for j in range(_L):
            r = g * _L + j
            accs = []
            for q in range(4):
                sl = pl.ds(q * _L, _L)
                accs.append(ue_v[r, sl] * ie_v[r, sl])
            p = (accs[0] + accs[1]) + (accs[2] + accs[3])
            res = jnp.where(lanes == j, jnp.sum(p), res)
        sl = pl.ds(g * _L, _L)
        out_v[sl] = res + ub_v[sl] + ib_v[sl]
        return carry

    lax.fori_loop(0, _BPW // _L, group, 0)
    pltpu.sync_copy(out_v, out_hbm.at[pl.ds(base, _BPW)])


def kernel(users, items, user_embeddings, item_embeddings, user_biases,
           item_biases):
    ub = user_biases.reshape(-1)
    ib = item_biases.reshape(-1)
    ue_rows, ubg = _gather_rows(users, user_embeddings, ub)
    ie_rows, ibg = _gather_rows(items, item_embeddings, ib)
    out = _combine(ue_rows, ubg, ie_rows, ibg)
    return out.reshape(_B, 1)
